# baseline (speedup 1.0000x reference)
import functools

import jax
import jax.numpy as jnp
from jax import lax
from jax.experimental import pallas as pl
from jax.experimental.pallas import tpu as pltpu
from jax.experimental.pallas import tpu_sc as plsc

_NC = 2
_NS = 16
_NW = _NC * _NS
_L = 16

_N_CELLS = 16384
_D = 64
_BPW = _N_CELLS // _NW
_IDX_CHUNK = 128
_NCHUNK = _BPW // _IDX_CHUNK


def _body(z_hbm, idx_hbm, table_hbm, out_hbm, idx_v, acc_v, rows_v, sem):
    wid = lax.axis_index("s") * _NC + lax.axis_index("c")
    base = wid * _BPW

    pltpu.sync_copy(idx_hbm.at[pl.ds(wid * _NCHUNK, _NCHUNK)], idx_v)

    copies = [
        pltpu.async_copy(
            table_hbm.at[idx_v.at[j]],
            rows_v.at[pl.ds(j * _IDX_CHUNK, _IDX_CHUNK)],
            sem,
        )
        for j in range(_NCHUNK)
    ]

    pltpu.sync_copy(z_hbm.at[pl.ds(base, _BPW)], acc_v)

    for cp in copies:
        cp.wait()

    def row_add(r, carry):
        for c in range(_D // _L):
            sl = pl.ds(c * _L, _L)
            plsc.addupdate(acc_v.at[r, sl], rows_v[r, sl])
        return carry

    lax.fori_loop(0, _BPW, row_add, 0, unroll=8)

    pltpu.sync_copy(acc_v, out_hbm.at[pl.ds(base, _BPW)])


@jax.jit
def kernel(z_bio, batch_ids, emb_weight):
    idx2d = batch_ids.astype(jnp.int32).reshape(_NW * _NCHUNK, _IDX_CHUNK)
    mesh = plsc.VectorSubcoreMesh(
        core_axis_name="c", subcore_axis_name="s",
        num_cores=_NC, num_subcores=_NS,
    )
    f = pl.kernel(
        _body,
        out_type=jax.ShapeDtypeStruct((_N_CELLS, _D), jnp.float32),
        mesh=mesh,
        scratch_types=[
            pltpu.VMEM((_NCHUNK, _IDX_CHUNK), jnp.int32),
            pltpu.VMEM((_BPW, _D), jnp.float32),
            pltpu.VMEM((_BPW, _D), jnp.float32),
            pltpu.SemaphoreType.DMA,
        ],
        compiler_params=pltpu.CompilerParams(use_tc_tiling_on_sc=False),
    )
    return f(z_bio, idx2d, emb_weight)
